# baseline (speedup 1.0000x reference)
import jax
import jax.numpy as jnp
from jax import lax
from jax.experimental import pallas as pl
from jax.experimental.pallas import tpu as pltpu
from jax.experimental.pallas import tpu_sc as plsc

N_FIELDS = 26
VOCAB = 100000
DIM = 32
BATCH = 16384
NPAIR = N_FIELDS * DIM

NC, NS, L = 2, 16, 16
NW = NC * NS
PPW = NPAIR // NW
BULK = 99968
TAIL = VOCAB + 1 - BULK
ROWBUF = BULK + 128
QB = 4096
CHUNKS = [(0, 25088), (25088, 25088), (50176, 25088), (75264, 24704)]


def _body(tables_hbm, tail_hbm, idx_hbm, wsplat_hbm, out_hbm,
          row_v, idx_v, col_v, wsplat_v, rsem, ssem):
    wid = lax.axis_index("s") * NC + lax.axis_index("c")
    p0 = wid * PPW
    pltpu.sync_copy(wsplat_hbm, wsplat_v)
    for s in range(2):
        pltpu.async_copy(col_v.at[s], out_hbm.at[p0, pl.ds(s * QB, QB)], ssem)

    def do_pair(i, carry):
        p = p0 + i
        f = p // DIM
        d = p - f * DIM
        for off, w in CHUNKS:
            pltpu.async_copy(tables_hbm.at[f, d, pl.ds(off, w)],
                             row_v.at[pl.ds(off, w)], rsem)
        pltpu.async_copy(tail_hbm.at[f, d], row_v.at[pl.ds(BULK, 128)], rsem)
        pltpu.async_copy(idx_hbm.at[f], idx_v, rsem)
        for off, w in CHUNKS:
            pltpu.make_async_copy(tables_hbm.at[f, d, pl.ds(off, w)],
                                  row_v.at[pl.ds(off, w)], rsem).wait()
        pltpu.make_async_copy(tail_hbm.at[f, d],
                              row_v.at[pl.ds(BULK, 128)], rsem).wait()
        pltpu.make_async_copy(idx_hbm.at[f], idx_v, rsem).wait()
        wv = wsplat_v[f]

        for k in range(4):
            s = k % 2
            pltpu.make_async_copy(col_v.at[s],
                                  out_hbm.at[p, pl.ds(k * QB, QB)],
                                  ssem).wait()

            @plsc.parallel_loop(0, QB // L, unroll=8)
            def gath(c, k=k, s=s):
                o = c * L
                v = idx_v[pl.ds(k * QB + o, L)]
                g = jnp.where((v >= 0) & (v < VOCAB), v + 1, 0)
                col_v[s, pl.ds(o, L)] = plsc.load_gather(row_v, [g]) * wv
            pltpu.async_copy(col_v.at[s], out_hbm.at[p, pl.ds(k * QB, QB)],
                             ssem)
        return carry

    lax.fori_loop(0, PPW, do_pair, 0)
    for s in range(2):
        pltpu.make_async_copy(col_v.at[s],
                              out_hbm.at[p0, pl.ds(s * QB, QB)], ssem).wait()


def kernel(indices, tables, weights):
    tables_t = jnp.transpose(tables, (0, 2, 1))
    idx_t = indices.T
    wsplat = jnp.broadcast_to(weights[:, None], (N_FIELDS, L))
    tail_pad = jnp.pad(tables_t[:, :, BULK:],
                       ((0, 0), (0, 0), (0, 128 - TAIL)))
    run = pl.kernel(
        _body,
        out_type=jax.ShapeDtypeStruct((NPAIR, BATCH), jnp.float32),
        mesh=plsc.VectorSubcoreMesh(core_axis_name="c", subcore_axis_name="s",
                                    num_cores=NC, num_subcores=NS),
        compiler_params=pltpu.CompilerParams(needs_layout_passes=False,
                                             disable_bounds_checks=True),
        scratch_types=[
            pltpu.VMEM((ROWBUF,), jnp.float32),
            pltpu.VMEM((BATCH,), jnp.int32),
            pltpu.VMEM((2, QB), jnp.float32),
            pltpu.VMEM((N_FIELDS, L), jnp.float32),
            pltpu.SemaphoreType.DMA,
            pltpu.SemaphoreType.DMA,
        ],
    )
    out_t = run(tables_t, tail_pad, idx_t, wsplat)
    return out_t.T


# speedup vs baseline: 1.0447x; 1.0447x over previous
import jax
import jax.numpy as jnp
from jax import lax
from jax.experimental import pallas as pl
from jax.experimental.pallas import tpu as pltpu
from jax.experimental.pallas import tpu_sc as plsc

N_FIELDS = 26
VOCAB = 100000
DIM = 32
BATCH = 16384
NPAIR = N_FIELDS * DIM

NC, NS, L = 2, 16, 16
NW = NC * NS
PPW = NPAIR // NW
NVEC = BATCH // L

C = 33408
BULK = 99968
TAIL = VOCAB + 1 - BULK
C3W = BULK - 2 * C
QB = 4096


def _fire_chunk(tables_hbm, tail_hbm, buf, f, d, c, sem):
    if c < 2:
        pltpu.async_copy(tables_hbm.at[f, d, pl.ds(c * C, C)],
                         buf.at[pl.ds(0, C)], sem)
    else:
        pltpu.async_copy(tables_hbm.at[f, d, pl.ds(2 * C, C3W)],
                         buf.at[pl.ds(0, C3W)], sem)
        pltpu.async_copy(tail_hbm.at[f, d], buf.at[pl.ds(C3W, 128)], sem)


def _wait_chunk(tables_hbm, tail_hbm, buf, f, d, c, sem):
    if c < 2:
        pltpu.make_async_copy(tables_hbm.at[f, d, pl.ds(c * C, C)],
                              buf.at[pl.ds(0, C)], sem).wait()
    else:
        pltpu.make_async_copy(tables_hbm.at[f, d, pl.ds(2 * C, C3W)],
                              buf.at[pl.ds(0, C3W)], sem).wait()
        pltpu.make_async_copy(tail_hbm.at[f, d],
                              buf.at[pl.ds(C3W, 128)], sem).wait()


def _body(tables_hbm, tail_hbm, idx_hbm, wsplat_hbm, out_hbm,
          bufx, bufy, idx_v, col_v, wsplat_v, rsem, ssem):
    wid = lax.axis_index("s") * NC + lax.axis_index("c")
    p0 = wid * PPW
    f0 = p0 // DIM
    d0 = p0 - f0 * DIM
    pltpu.sync_copy(wsplat_hbm, wsplat_v)
    for k in range(4):
        pltpu.async_copy(col_v.at[pl.ds(k * QB, QB)],
                         out_hbm.at[p0, pl.ds(k * QB, QB)], ssem)
    _fire_chunk(tables_hbm, tail_hbm, bufx, f0, d0, 0, rsem)
    _fire_chunk(tables_hbm, tail_hbm, bufy, f0, d0, 1, rsem)

    def do_pair(i, carry):
        p = p0 + i
        f = p // DIM
        d = p - f * DIM
        pn = p0 + jnp.minimum(i + 1, PPW - 1)
        fn = pn // DIM
        dn = pn - fn * DIM
        pltpu.async_copy(idx_hbm.at[f], idx_v, rsem)
        wv = wsplat_v[f]

        for k in range(4):
            pltpu.make_async_copy(col_v.at[pl.ds(k * QB, QB)],
                                  out_hbm.at[p, pl.ds(k * QB, QB)],
                                  ssem).wait()
        pltpu.make_async_copy(idx_hbm.at[f], idx_v, rsem).wait()

        @plsc.parallel_loop(0, NVEC, unroll=8)
        def remap(c):
            v = idx_v[pl.ds(c * L, L)]
            idx_v[pl.ds(c * L, L)] = jnp.where(
                (v >= 0) & (v < VOCAB), v + 1, 0)

        _wait_chunk(tables_hbm, tail_hbm, bufx, f, d, 0, rsem)

        @plsc.parallel_loop(0, NVEC, unroll=8)
        def pass1(c):
            g = idx_v[pl.ds(c * L, L)]
            x = plsc.load_gather(bufx, [g], mask=g < C)
            col_v[pl.ds(c * L, L)] = x * wv

        _wait_chunk(tables_hbm, tail_hbm, bufy, f, d, 1, rsem)
        _fire_chunk(tables_hbm, tail_hbm, bufx, f, d, 2, rsem)

        @plsc.parallel_loop(0, NVEC, unroll=8)
        def pass2(c):
            g = idx_v[pl.ds(c * L, L)]
            m = (g >= C) & (g < 2 * C)
            x = plsc.load_gather(bufy, [g - C], mask=m)
            cur = col_v[pl.ds(c * L, L)]
            col_v[pl.ds(c * L, L)] = jnp.where(m, x * wv, cur)

        _wait_chunk(tables_hbm, tail_hbm, bufx, f, d, 2, rsem)
        _fire_chunk(tables_hbm, tail_hbm, bufy, fn, dn, 0, rsem)

        for k in range(4):
            @plsc.parallel_loop(0, QB // L, unroll=8)
            def pass3(c, k=k):
                o = k * QB + c * L
                g = idx_v[pl.ds(o, L)]
                m = g >= 2 * C
                x = plsc.load_gather(bufx, [g - 2 * C], mask=m)
                cur = col_v[pl.ds(o, L)]
                col_v[pl.ds(o, L)] = jnp.where(m, x * wv, cur)

            pltpu.async_copy(col_v.at[pl.ds(k * QB, QB)],
                             out_hbm.at[p, pl.ds(k * QB, QB)], ssem)

        _fire_chunk(tables_hbm, tail_hbm, bufx, fn, dn, 1, rsem)
        return carry

    lax.fori_loop(0, PPW, do_pair, 0)
    pe = p0 + PPW - 1
    fe = pe // DIM
    de = pe - fe * DIM
    for k in range(4):
        pltpu.make_async_copy(col_v.at[pl.ds(k * QB, QB)],
                              out_hbm.at[pe, pl.ds(k * QB, QB)], ssem).wait()
    _wait_chunk(tables_hbm, tail_hbm, bufy, fe, de, 0, rsem)
    _wait_chunk(tables_hbm, tail_hbm, bufx, fe, de, 1, rsem)


def kernel(indices, tables, weights):
    tables_t = jnp.transpose(tables, (0, 2, 1))
    idx_t = indices.T
    wsplat = jnp.broadcast_to(weights[:, None], (N_FIELDS, L))
    tail_pad = jnp.pad(tables_t[:, :, BULK:],
                       ((0, 0), (0, 0), (0, 128 - TAIL)))
    run = pl.kernel(
        _body,
        out_type=jax.ShapeDtypeStruct((NPAIR, BATCH), jnp.float32),
        mesh=plsc.VectorSubcoreMesh(core_axis_name="c", subcore_axis_name="s",
                                    num_cores=NC, num_subcores=NS),
        compiler_params=pltpu.CompilerParams(needs_layout_passes=False,
                                             disable_bounds_checks=True),
        scratch_types=[
            pltpu.VMEM((C,), jnp.float32),
            pltpu.VMEM((C,), jnp.float32),
            pltpu.VMEM((BATCH,), jnp.int32),
            pltpu.VMEM((BATCH,), jnp.float32),
            pltpu.VMEM((N_FIELDS, L), jnp.float32),
            pltpu.SemaphoreType.DMA,
            pltpu.SemaphoreType.DMA,
        ],
    )
    out_t = run(tables_t, tail_pad, idx_t, wsplat)
    return out_t.T
